# baseline (speedup 1.0000x reference)
import functools

import jax
import jax.numpy as jnp
from jax import lax
from jax.experimental import pallas as pl
from jax.experimental.pallas import tpu as pltpu
from jax.experimental.pallas import tpu_sc as plsc

_CONF = 0.5
_MIN_SZ = 0.01
_MIN_AR = 0.2
_MAX_AR = 5.0
_NMS_T = 0.5
_MAXDET = 100
_NEG = -1e9

_NSUB = 16
_L = 16
_PER_W = 320
_NCH = _PER_W // _L
_PAD = _NSUB * _PER_W
_NC4 = 4
_RW = 32
_TBL = _NSUB * _RW
_FTBL = _NSUB * _L


def _splat_f(x):
    return jnp.full((_L,), x, jnp.float32)


def _splat_i(x):
    return jnp.full((_L,), x, jnp.int32)


def _perm(v, idx):
    return v.at[idx].get(mode="promise_in_bounds")


def _nms_body(x1_hbm, y1_hbm, x2_hbm, y2_hbm, scores_hbm, out_hbm,
              x1_v, y1_v, x2_v, y2_v, area_v, sact_v,
              stage_v, allrec_v, frec_v, outbuf_v, shared_rec, shared_rec2):
    cid = lax.axis_index("c")
    sid = lax.axis_index("s")
    base = sid * _PER_W
    lane = lax.broadcasted_iota(jnp.int32, (_L,), 0)

    def lex_gt(v1, i1, v2, i2):
        return (v1 > v2) | ((v1 == v2) & (i1 < i2))

    def lex_reduce(val, idx):
        for k in (8, 4, 2, 1):
            p = jnp.bitwise_xor(lane, k)
            pv = _perm(val, p)
            pi = _perm(idx, p)
            upd = lex_gt(pv, pi, val, idx)
            val = jnp.where(upd, pv, val)
            idx = jnp.where(upd, pi, idx)
        return val, idx

    def lex3_reduce(val, key, pay):
        for k in (8, 4, 2, 1):
            p = jnp.bitwise_xor(lane, k)
            pv = _perm(val, p)
            pk = _perm(key, p)
            pp = _perm(pay, p)
            upd = lex_gt(pv, pk, val, key)
            val = jnp.where(upd, pv, val)
            key = jnp.where(upd, pk, key)
            pay = jnp.where(upd, pp, pay)
        return val, key, pay

    pltpu.sync_copy(x1_hbm.at[pl.ds(base, _PER_W)], x1_v)
    pltpu.sync_copy(y1_hbm.at[pl.ds(base, _PER_W)], y1_v)
    pltpu.sync_copy(x2_hbm.at[pl.ds(base, _PER_W)], x2_v)
    pltpu.sync_copy(y2_hbm.at[pl.ds(base, _PER_W)], y2_v)
    pltpu.sync_copy(scores_hbm.at[pl.ds(base, _PER_W)], sact_v)

    for c in range(_NCH):
        sl = pl.ds(c * _L, _L)
        x1 = jnp.clip(x1_v[sl], 0.0, 1.0)
        y1 = jnp.clip(y1_v[sl], 0.0, 1.0)
        x2 = jnp.clip(x2_v[sl], 0.0, 1.0)
        y2 = jnp.clip(y2_v[sl], 0.0, 1.0)
        sc = sact_v[sl]
        w = x2 - x1
        h = y2 - y1
        valid = (sc > _CONF) & (w > _MIN_SZ) & (h > _MIN_SZ)
        aspect = w / (h + 1e-6)
        valid = valid & (aspect > _MIN_AR) & (aspect < _MAX_AR)
        x1 = jnp.where(valid, x1, 0.0)
        y1 = jnp.where(valid, y1, 0.0)
        x2 = jnp.where(valid, x2, 0.0)
        y2 = jnp.where(valid, y2, 0.0)
        x1_v[sl] = x1
        y1_v[sl] = y1
        x2_v[sl] = x2
        y2_v[sl] = y2
        area_v[sl] = (x2 - x1) * (y2 - y1)
        sact_v[sl] = jnp.where(valid, sc, _NEG)

    def box_at(iv):
        return (plsc.load_gather(x1_v, [iv]), plsc.load_gather(y1_v, [iv]),
                plsc.load_gather(x2_v, [iv]), plsc.load_gather(y2_v, [iv]),
                plsc.load_gather(area_v, [iv]))

    _EMPTY4 = tuple(
        x for _ in range(_NC4) for x in (_splat_f(_NEG), _splat_i(0)))

    def acc_top4(v, i, st):
        b1v, b1i, b2v, b2i, b3v, b3i, b4v, b4i = st
        gt1 = v > b1v
        gt2 = v > b2v
        gt3 = v > b3v
        gt4 = v > b4v
        n1v = jnp.where(gt1, v, b1v)
        n1i = jnp.where(gt1, i, b1i)
        n2v = jnp.where(gt1, b1v, jnp.where(gt2, v, b2v))
        n2i = jnp.where(gt1, b1i, jnp.where(gt2, i, b2i))
        n3v = jnp.where(gt2, b2v, jnp.where(gt3, v, b3v))
        n3i = jnp.where(gt2, b2i, jnp.where(gt3, i, b3i))
        n4v = jnp.where(gt3, b3v, jnp.where(gt4, v, b4v))
        n4i = jnp.where(gt3, b3i, jnp.where(gt4, i, b4i))
        return (n1v, n1i, n2v, n2i, n3v, n3i, n4v, n4i)

    def _ce(av, ai, bv, bi):
        sw = lex_gt(bv, bi, av, ai)
        return (jnp.where(sw, bv, av), jnp.where(sw, bi, ai),
                jnp.where(sw, av, bv), jnp.where(sw, ai, bi))

    def top4_merge(st):
        b1v, b1i, b2v, b2i, b3v, b3i, b4v, b4i = st
        for k in (8, 4, 2, 1):
            p = jnp.bitwise_xor(lane, k)
            c1v, c1i = _perm(b1v, p), _perm(b1i, p)
            c2v, c2i = _perm(b2v, p), _perm(b2i, p)
            c3v, c3i = _perm(b3v, p), _perm(b3i, p)
            c4v, c4i = _perm(b4v, p), _perm(b4i, p)
            d1v, d1i = jnp.where(lex_gt(b1v, b1i, c4v, c4i), b1v, c4v), \
                jnp.where(lex_gt(b1v, b1i, c4v, c4i), b1i, c4i)
            d2v, d2i = jnp.where(lex_gt(b2v, b2i, c3v, c3i), b2v, c3v), \
                jnp.where(lex_gt(b2v, b2i, c3v, c3i), b2i, c3i)
            d3v, d3i = jnp.where(lex_gt(b3v, b3i, c2v, c2i), b3v, c2v), \
                jnp.where(lex_gt(b3v, b3i, c2v, c2i), b3i, c2i)
            d4v, d4i = jnp.where(lex_gt(b4v, b4i, c1v, c1i), b4v, c1v), \
                jnp.where(lex_gt(b4v, b4i, c1v, c1i), b4i, c1i)
            d1v, d1i, d3v, d3i = _ce(d1v, d1i, d3v, d3i)
            d2v, d2i, d4v, d4i = _ce(d2v, d2i, d4v, d4i)
            b1v, b1i, b2v, b2i = _ce(d1v, d1i, d2v, d2i)
            b3v, b3i, b4v, b4i = _ce(d3v, d3i, d4v, d4i)
        return (b1v, b1i, b2v, b2i, b3v, b3i, b4v, b4i)

    def _ce3(a, b):
        sw = lex_gt(b[0], b[1], a[0], a[1])
        hi = tuple(jnp.where(sw, y, x) for x, y in zip(a, b))
        lo = tuple(jnp.where(sw, x, y) for x, y in zip(a, b))
        return hi, lo

    def top4_merge3(t):
        for k in (8, 4, 2, 1):
            p = jnp.bitwise_xor(lane, k)
            c = [tuple(_perm(x, p) for x in tj) for tj in t]
            d = []
            for j in range(_NC4):
                a, b = t[j], c[3 - j]
                ge = lex_gt(a[0], a[1], b[0], b[1])
                d.append(tuple(jnp.where(ge, x, y) for x, y in zip(a, b)))
            d0, d2 = _ce3(d[0], d[2])
            d1, d3 = _ce3(d[1], d[3])
            t0, t1 = _ce3(d0, d1)
            t2, t3 = _ce3(d2, d3)
            t = [t0, t1, t2, t3]
        return t

    def build_vreg(pairs):
        terms = [jnp.where(lane == f, v, 0.0) for f, v in pairs]
        while len(terms) > 1:
            nxt = [terms[j] + terms[j + 1] for j in range(0, len(terms) - 1, 2)]
            if len(terms) % 2:
                nxt.append(terms[-1])
            terms = nxt
        return terms[0]

    def iou_gt(ax1, ay1, ax2, ay2, aar, bx1, by1, bx2, by2, bar):
        iw = jnp.maximum(jnp.minimum(ax2, bx2) - jnp.maximum(ax1, bx1), 0.0)
        ih = jnp.maximum(jnp.minimum(ay2, by2) - jnp.maximum(ay1, by1), 0.0)
        inter = iw * ih
        return inter / (aar + bar - inter + 1e-9) > _NMS_T

    def sweep_filler():
        bestv = _splat_f(_NEG)
        besti = _splat_i(0)
        for c in range(_NCH):
            i = lane + c * _L
            v = sact_v[pl.ds(c * _L, _L)]
            v = jnp.where((v > -1.5) & (v < 0.0), -v, _NEG)
            upd = v > bestv
            besti = jnp.where(upd, i, besti)
            bestv = jnp.where(upd, v, bestv)
        return lex_reduce(bestv, besti)

    def round_body(carry):
        (r, rnd, s1v, s1i, s2v, s2i, s3v, s3i, s4v, s4i) = carry
        parity = jnp.bitwise_and(rnd, 1)

        pairs_a, pairs_b = [], []
        for j, (sv, siv) in enumerate(((s1v, s1i), (s2v, s2i),
                                       (s3v, s3i), (s4v, s4i))):
            bx1, by1, bx2, by2, bar = box_at(siv)
            gvf = (siv + base).astype(jnp.float32)
            for f, val in enumerate((sv, gvf, bx1, by1, bx2, by2, bar)):
                off = j * 7 + f
                (pairs_a if off < _L else pairs_b).append((off % _L, val))
        stage_v[pl.ds(0, _L)] = build_vreg(pairs_a)
        stage_v[pl.ds(_L, _L)] = build_vreg(pairs_b)
        pltpu.sync_copy(stage_v,
                        shared_rec.at[pl.ds(parity * _TBL + sid * _RW, _RW)])
        plsc.subcore_barrier()
        pltpu.sync_copy(shared_rec.at[pl.ds(parity * _TBL, _TBL)], allrec_v)

        tin = []
        for j in range(_NC4):
            oj = lane * _RW + j * 7
            mj = plsc.load_gather(allrec_v, [oj])
            gj = plsc.load_gather(allrec_v, [oj + 1]).astype(jnp.int32)
            tin.append((mj, gj, oj))
        tout = top4_merge3(tin)
        cand = []
        for j in range(_NC4):
            gv, gk, o = tout[j]
            fx1 = plsc.load_gather(allrec_v, [o + 2])
            fy1 = plsc.load_gather(allrec_v, [o + 3])
            fx2 = plsc.load_gather(allrec_v, [o + 4])
            fy2 = plsc.load_gather(allrec_v, [o + 5])
            far = plsc.load_gather(allrec_v, [o + 6])
            cand.append((gv, gk, fx1, fy1, fx2, fy2, far))

        (m1, g1, wx1, wy1, wx2, wy2, wa1) = cand[0]
        (m2, g2, cx1, cy1, cx2, cy2, wa2) = cand[1]
        (m3, g3, dx1, dy1, dx2, dy2, wa3) = cand[2]
        (m4, g4, ex1, ey1, ex2, ey2, wa4) = cand[3]

        use1 = m1 > 0.0
        use1_s = jnp.any(use1)

        k2 = (m2 > 0.0) & jnp.logical_not(
            iou_gt(wx1, wy1, wx2, wy2, wa1, cx1, cy1, cx2, cy2, wa2))
        ov13 = iou_gt(wx1, wy1, wx2, wy2, wa1, dx1, dy1, dx2, dy2, wa3)
        ov23 = iou_gt(cx1, cy1, cx2, cy2, wa2, dx1, dy1, dx2, dy2, wa3)
        k3 = (m3 > 0.0) & jnp.logical_not(ov13 | (k2 & ov23))
        ov14 = iou_gt(wx1, wy1, wx2, wy2, wa1, ex1, ey1, ex2, ey2, wa4)
        ov24 = iou_gt(cx1, cy1, cx2, cy2, wa2, ex1, ey1, ex2, ey2, wa4)
        ov34 = iou_gt(dx1, dy1, dx2, dy2, wa3, ex1, ey1, ex2, ey2, wa4)
        k4 = (m4 > 0.0) & jnp.logical_not(ov14 | (k2 & ov24) | (k3 & ov34))
        k2_s = jnp.any(k2)
        k3_s = jnp.any(k3)
        k4_s = jnp.any(k4)

        def chunk_body(c, st):
            sl = pl.ds(c * _L, _L)
            x1 = x1_v[sl]
            y1 = y1_v[sl]
            x2 = x2_v[sl]
            y2 = y2_v[sl]
            ar = area_v[sl]
            sa = sact_v[sl]
            li = lane + c * _L
            gi = li + _splat_i(base)
            h1 = use1 & (iou_gt(x1, y1, x2, y2, ar, wx1, wy1, wx2, wy2, wa1)
                         | (gi == g1))
            h2 = k2 & (iou_gt(x1, y1, x2, y2, ar, cx1, cy1, cx2, cy2, wa2)
                       | (gi == g2))
            h3 = k3 & (iou_gt(x1, y1, x2, y2, ar, dx1, dy1, dx2, dy2, wa3)
                       | (gi == g3))
            h4 = k4 & (iou_gt(x1, y1, x2, y2, ar, ex1, ey1, ex2, ey2, wa4)
                       | (gi == g4))
            toneg = (use1 & (gi == g1)) | (k2 & (gi == g2)) \
                | (k3 & (gi == g3)) | (k4 & (gi == g4))
            hit = (sa > 0.0) & (h1 | h2 | h3 | h4)
            sa_new = jnp.where(hit, jnp.where(toneg, _NEG, -sa), sa)
            sact_v[sl] = sa_new
            return acc_top4(sa_new, li, st)

        st = lax.fori_loop(0, _NCH, chunk_body, _EMPTY4)

        k2_i = k2_s.astype(jnp.int32)
        k3_i = k3_s.astype(jnp.int32)
        k4_i = k4_s.astype(jnp.int32)
        pos2 = r + 1
        pos3 = r + 1 + k2_i
        pos4 = r + 1 + k2_i + k3_i

        @pl.when(use1_s & (sid == 0))
        def _out1():
            row = build_vreg([(0, wx1), (1, wy1), (2, wx2), (3, wy2), (4, m1)])
            plsc.store_scatter(outbuf_v, [_splat_i(r * _L) + lane], row)

        @pl.when(k2_s & (pos2 < _MAXDET) & (sid == 0))
        def _out2():
            row = build_vreg([(0, cx1), (1, cy1), (2, cx2), (3, cy2), (4, m2)])
            plsc.store_scatter(outbuf_v, [_splat_i(pos2 * _L) + lane], row)

        @pl.when(k3_s & (pos3 < _MAXDET) & (sid == 0))
        def _out3():
            row = build_vreg([(0, dx1), (1, dy1), (2, dx2), (3, dy2), (4, m3)])
            plsc.store_scatter(outbuf_v, [_splat_i(pos3 * _L) + lane], row)

        @pl.when(k4_s & (pos4 < _MAXDET) & (sid == 0))
        def _out4():
            row = build_vreg([(0, ex1), (1, ey1), (2, ex2), (3, ey2), (4, m4)])
            plsc.store_scatter(outbuf_v, [_splat_i(pos4 * _L) + lane], row)

        @pl.when(jnp.logical_not(use1_s))
        def _phase23():
            fmv, fiv = sweep_filler()
            fbx1, fby1, fbx2, fby2, _ = box_at(fiv)
            fgv = (fiv + base).astype(jnp.float32)
            rec = build_vreg([(0, fmv), (1, fgv), (2, fbx1), (3, fby1),
                              (4, fbx2), (5, fby2)])
            stage_v[pl.ds(0, _L)] = rec
            pltpu.sync_copy(
                stage_v.at[pl.ds(0, _L)],
                shared_rec2.at[pl.ds(parity * _FTBL + sid * _L, _L)])
            plsc.subcore_barrier()
            pltpu.sync_copy(shared_rec2.at[pl.ds(parity * _FTBL, _FTBL)],
                            frec_v)
            fcol = plsc.load_gather(frec_v, [lane * _L])
            gfv, widf = lex_reduce(fcol, lane)
            usef = gfv > 0.0

            fo = widf * _L
            fidxv = plsc.load_gather(frec_v, [fo + 1]).astype(jnp.int32)
            ox1 = plsc.load_gather(frec_v, [fo + 2])
            oy1 = plsc.load_gather(frec_v, [fo + 3])
            ox2 = plsc.load_gather(frec_v, [fo + 4])
            oy2 = plsc.load_gather(frec_v, [fo + 5])

            lidxv = fidxv - base
            owner = (lidxv >= 0) & (lidxv < _PER_W)
            lclampv = jnp.clip(lidxv, 0, _PER_W - 1)
            plsc.store_scatter(sact_v, [lclampv], _splat_f(_NEG),
                               mask=(lane == 0) & usef & owner)

            @pl.when(sid == 0)
            def _out23():
                zero = _splat_f(0.0)
                row = build_vreg([
                    (0, jnp.where(usef, ox1, zero)),
                    (1, jnp.where(usef, oy1, zero)),
                    (2, jnp.where(usef, ox2, zero)),
                    (3, jnp.where(usef, oy2, zero)),
                    (4, _splat_f(_NEG))])
                plsc.store_scatter(outbuf_v, [_splat_i(r * _L) + lane], row)

        nst = top4_merge(st)
        dr = jnp.where(use1_s, 1 + k2_i + k3_i + k4_i, 1).astype(jnp.int32)
        return (r + dr, rnd + 1) + nst

    st0 = _EMPTY4
    for c in range(_NCH):
        st0 = acc_top4(sact_v[pl.ds(c * _L, _L)], lane + c * _L, st0)
    init = (jnp.int32(0), jnp.int32(0)) + top4_merge(st0)
    lax.while_loop(lambda c: c[0] < _MAXDET, round_body, init)

    @pl.when((sid == 0) & (cid == 0))
    def _flush():
        pltpu.sync_copy(outbuf_v, out_hbm)


def kernel(boxes, scores):
    n = boxes.shape[0]
    boxes_p = jnp.zeros((_PAD, 4), jnp.float32).at[:n].set(boxes)
    scores_p = jnp.full((_PAD,), -1.0, jnp.float32).at[:n].set(scores)
    cx1, cy1, cx2, cy2 = (boxes_p[:, j] for j in range(4))

    mesh = plsc.VectorSubcoreMesh(core_axis_name="c", subcore_axis_name="s")
    run = functools.partial(
        pl.kernel,
        out_type=jax.ShapeDtypeStruct((_MAXDET * _L,), jnp.float32),
        mesh=mesh,
        compiler_params=pltpu.CompilerParams(needs_layout_passes=False),
        scratch_types=[
            pltpu.VMEM((_PER_W,), jnp.float32),
            pltpu.VMEM((_PER_W,), jnp.float32),
            pltpu.VMEM((_PER_W,), jnp.float32),
            pltpu.VMEM((_PER_W,), jnp.float32),
            pltpu.VMEM((_PER_W,), jnp.float32),
            pltpu.VMEM((_PER_W,), jnp.float32),
            pltpu.VMEM((_RW,), jnp.float32),
            pltpu.VMEM((_TBL,), jnp.float32),
            pltpu.VMEM((_FTBL,), jnp.float32),
            pltpu.VMEM((_MAXDET * _L,), jnp.float32),
            pltpu.VMEM_SHARED((2 * _TBL,), jnp.float32),
            pltpu.VMEM_SHARED((2 * _FTBL,), jnp.float32),
        ],
    )(_nms_body)
    out = run(cx1, cy1, cx2, cy2, scores_p)
    return out.reshape(_MAXDET, _L)[:, :5]


# speedup vs baseline: 1.0459x; 1.0459x over previous
import functools

import jax
import jax.numpy as jnp
from jax import lax
from jax.experimental import pallas as pl
from jax.experimental.pallas import tpu as pltpu
from jax.experimental.pallas import tpu_sc as plsc

_CONF = 0.5
_MIN_SZ = 0.01
_MIN_AR = 0.2
_MAX_AR = 5.0
_NMS_T = 0.5
_MAXDET = 100
_NEG = -1e9

_NSUB = 16
_L = 16
_PER_W = 320
_NCH = _PER_W // _L
_PAD = _NSUB * _PER_W
_NC4 = 4
_RW = 32
_TBL = _NSUB * _RW
_FTBL = _NSUB * _L


def _splat_f(x):
    return jnp.full((_L,), x, jnp.float32)


def _splat_i(x):
    return jnp.full((_L,), x, jnp.int32)


def _perm(v, idx):
    return v.at[idx].get(mode="promise_in_bounds")


def _nms_body(x1_hbm, y1_hbm, x2_hbm, y2_hbm, scores_hbm, out_hbm,
              x1_v, y1_v, x2_v, y2_v, area_v, sact_v,
              stage_v, allrec_v, frec_v, outbuf_v, shared_rec, shared_rec2):
    cid = lax.axis_index("c")
    sid = lax.axis_index("s")
    base = sid * _PER_W
    lane = lax.broadcasted_iota(jnp.int32, (_L,), 0)

    def lex_gt(v1, i1, v2, i2):
        return (v1 > v2) | ((v1 == v2) & (i1 < i2))

    def lex_reduce(val, idx):
        for k in (8, 4, 2, 1):
            p = jnp.bitwise_xor(lane, k)
            pv = _perm(val, p)
            pi = _perm(idx, p)
            upd = lex_gt(pv, pi, val, idx)
            val = jnp.where(upd, pv, val)
            idx = jnp.where(upd, pi, idx)
        return val, idx

    pltpu.sync_copy(x1_hbm.at[pl.ds(base, _PER_W)], x1_v)
    pltpu.sync_copy(y1_hbm.at[pl.ds(base, _PER_W)], y1_v)
    pltpu.sync_copy(x2_hbm.at[pl.ds(base, _PER_W)], x2_v)
    pltpu.sync_copy(y2_hbm.at[pl.ds(base, _PER_W)], y2_v)
    pltpu.sync_copy(scores_hbm.at[pl.ds(base, _PER_W)], sact_v)

    for c in range(_NCH):
        sl = pl.ds(c * _L, _L)
        x1 = jnp.clip(x1_v[sl], 0.0, 1.0)
        y1 = jnp.clip(y1_v[sl], 0.0, 1.0)
        x2 = jnp.clip(x2_v[sl], 0.0, 1.0)
        y2 = jnp.clip(y2_v[sl], 0.0, 1.0)
        sc = sact_v[sl]
        w = x2 - x1
        h = y2 - y1
        valid = (sc > _CONF) & (w > _MIN_SZ) & (h > _MIN_SZ)
        aspect = w / (h + 1e-6)
        valid = valid & (aspect > _MIN_AR) & (aspect < _MAX_AR)
        x1 = jnp.where(valid, x1, 0.0)
        y1 = jnp.where(valid, y1, 0.0)
        x2 = jnp.where(valid, x2, 0.0)
        y2 = jnp.where(valid, y2, 0.0)
        x1_v[sl] = x1
        y1_v[sl] = y1
        x2_v[sl] = x2
        y2_v[sl] = y2
        area_v[sl] = (x2 - x1) * (y2 - y1)
        sact_v[sl] = jnp.where(valid, sc, _NEG)

    def box_at(iv):
        return (plsc.load_gather(x1_v, [iv]), plsc.load_gather(y1_v, [iv]),
                plsc.load_gather(x2_v, [iv]), plsc.load_gather(y2_v, [iv]),
                plsc.load_gather(area_v, [iv]))

    _EMPTY4 = tuple(
        x for _ in range(_NC4) for x in (_splat_f(_NEG), _splat_i(0)))

    def acc_top4(v, i, st):
        b1v, b1i, b2v, b2i, b3v, b3i, b4v, b4i = st
        gt1 = v > b1v
        gt2 = v > b2v
        gt3 = v > b3v
        gt4 = v > b4v
        n1v = jnp.where(gt1, v, b1v)
        n1i = jnp.where(gt1, i, b1i)
        n2v = jnp.where(gt1, b1v, jnp.where(gt2, v, b2v))
        n2i = jnp.where(gt1, b1i, jnp.where(gt2, i, b2i))
        n3v = jnp.where(gt2, b2v, jnp.where(gt3, v, b3v))
        n3i = jnp.where(gt2, b2i, jnp.where(gt3, i, b3i))
        n4v = jnp.where(gt3, b3v, jnp.where(gt4, v, b4v))
        n4i = jnp.where(gt3, b3i, jnp.where(gt4, i, b4i))
        return (n1v, n1i, n2v, n2i, n3v, n3i, n4v, n4i)

    def _ce(av, ai, bv, bi):
        sw = lex_gt(bv, bi, av, ai)
        return (jnp.where(sw, bv, av), jnp.where(sw, bi, ai),
                jnp.where(sw, av, bv), jnp.where(sw, ai, bi))

    def top4_merge(st):
        b1v, b1i, b2v, b2i, b3v, b3i, b4v, b4i = st
        for k in (8, 4, 2, 1):
            p = jnp.bitwise_xor(lane, k)
            c1v, c1i = _perm(b1v, p), _perm(b1i, p)
            c2v, c2i = _perm(b2v, p), _perm(b2i, p)
            c3v, c3i = _perm(b3v, p), _perm(b3i, p)
            c4v, c4i = _perm(b4v, p), _perm(b4i, p)
            d1v, d1i = jnp.where(lex_gt(b1v, b1i, c4v, c4i), b1v, c4v), \
                jnp.where(lex_gt(b1v, b1i, c4v, c4i), b1i, c4i)
            d2v, d2i = jnp.where(lex_gt(b2v, b2i, c3v, c3i), b2v, c3v), \
                jnp.where(lex_gt(b2v, b2i, c3v, c3i), b2i, c3i)
            d3v, d3i = jnp.where(lex_gt(b3v, b3i, c2v, c2i), b3v, c2v), \
                jnp.where(lex_gt(b3v, b3i, c2v, c2i), b3i, c2i)
            d4v, d4i = jnp.where(lex_gt(b4v, b4i, c1v, c1i), b4v, c1v), \
                jnp.where(lex_gt(b4v, b4i, c1v, c1i), b4i, c1i)
            d1v, d1i, d3v, d3i = _ce(d1v, d1i, d3v, d3i)
            d2v, d2i, d4v, d4i = _ce(d2v, d2i, d4v, d4i)
            b1v, b1i, b2v, b2i = _ce(d1v, d1i, d2v, d2i)
            b3v, b3i, b4v, b4i = _ce(d3v, d3i, d4v, d4i)
        return (b1v, b1i, b2v, b2i, b3v, b3i, b4v, b4i)

    def _ce3(a, b):
        sw = lex_gt(b[0], b[1], a[0], a[1])
        hi = tuple(jnp.where(sw, y, x) for x, y in zip(a, b))
        lo = tuple(jnp.where(sw, x, y) for x, y in zip(a, b))
        return hi, lo

    def top4_merge3(t):
        for k in (8, 4, 2, 1):
            p = jnp.bitwise_xor(lane, k)
            c = [tuple(_perm(x, p) for x in tj) for tj in t]
            d = []
            for j in range(_NC4):
                a, b = t[j], c[3 - j]
                ge = lex_gt(a[0], a[1], b[0], b[1])
                d.append(tuple(jnp.where(ge, x, y) for x, y in zip(a, b)))
            d0, d2 = _ce3(d[0], d[2])
            d1, d3 = _ce3(d[1], d[3])
            t0, t1 = _ce3(d0, d1)
            t2, t3 = _ce3(d2, d3)
            t = [t0, t1, t2, t3]
        return t

    def build_vreg(pairs):
        terms = [jnp.where(lane == f, v, 0.0) for f, v in pairs]
        while len(terms) > 1:
            nxt = [terms[j] + terms[j + 1] for j in range(0, len(terms) - 1, 2)]
            if len(terms) % 2:
                nxt.append(terms[-1])
            terms = nxt
        return terms[0]

    def iou_gt(ax1, ay1, ax2, ay2, aar, bx1, by1, bx2, by2, bar):
        iw = jnp.maximum(jnp.minimum(ax2, bx2) - jnp.maximum(ax1, bx1), 0.0)
        ih = jnp.maximum(jnp.minimum(ay2, by2) - jnp.maximum(ay1, by1), 0.0)
        inter = iw * ih
        return inter / (aar + bar - inter + 1e-9) > _NMS_T

    def sweep_filler():
        bestv = _splat_f(_NEG)
        besti = _splat_i(0)
        for c in range(_NCH):
            i = lane + c * _L
            v = sact_v[pl.ds(c * _L, _L)]
            v = jnp.where((v > -1.5) & (v < 0.0), -v, _NEG)
            upd = v > bestv
            besti = jnp.where(upd, i, besti)
            bestv = jnp.where(upd, v, bestv)
        return lex_reduce(bestv, besti)

    def round_body(carry):
        (r, rnd, s1v, s1i, s2v, s2i, s3v, s3i, s4v, s4i) = carry
        parity = jnp.bitwise_and(rnd, 1)

        pairs_a, pairs_b = [], []
        for j, (sv, siv) in enumerate(((s1v, s1i), (s2v, s2i),
                                       (s3v, s3i), (s4v, s4i))):
            bx1, by1, bx2, by2, bar = box_at(siv)
            gvf = (siv + base).astype(jnp.float32)
            for f, val in enumerate((sv, gvf, bx1, by1, bx2, by2, bar)):
                off = j * 7 + f
                (pairs_a if off < _L else pairs_b).append((off % _L, val))
        stage_v[pl.ds(0, _L)] = build_vreg(pairs_a)
        stage_v[pl.ds(_L, _L)] = build_vreg(pairs_b)
        pltpu.sync_copy(stage_v,
                        shared_rec.at[pl.ds(parity * _TBL + sid * _RW, _RW)])
        plsc.subcore_barrier()
        pltpu.sync_copy(shared_rec.at[pl.ds(parity * _TBL, _TBL)], allrec_v)

        tin = []
        for j in range(_NC4):
            oj = lane * _RW + j * 7
            mj = plsc.load_gather(allrec_v, [oj])
            gj = plsc.load_gather(allrec_v, [oj + 1]).astype(jnp.int32)
            tin.append((mj, gj, oj))
        tout = top4_merge3(tin)
        cand = []
        for j in range(_NC4):
            gv, gk, o = tout[j]
            fx1 = plsc.load_gather(allrec_v, [o + 2])
            fy1 = plsc.load_gather(allrec_v, [o + 3])
            fx2 = plsc.load_gather(allrec_v, [o + 4])
            fy2 = plsc.load_gather(allrec_v, [o + 5])
            far = plsc.load_gather(allrec_v, [o + 6])
            cand.append((gv, gk, fx1, fy1, fx2, fy2, far))

        (m1, g1, wx1, wy1, wx2, wy2, wa1) = cand[0]
        (m2, g2, cx1, cy1, cx2, cy2, wa2) = cand[1]
        (m3, g3, dx1, dy1, dx2, dy2, wa3) = cand[2]
        (m4, g4, ex1, ey1, ex2, ey2, wa4) = cand[3]

        use1 = m1 > 0.0
        use1_s = jnp.any(use1)

        k2 = (m2 > 0.0) & jnp.logical_not(
            iou_gt(wx1, wy1, wx2, wy2, wa1, cx1, cy1, cx2, cy2, wa2))
        ov13 = iou_gt(wx1, wy1, wx2, wy2, wa1, dx1, dy1, dx2, dy2, wa3)
        ov23 = iou_gt(cx1, cy1, cx2, cy2, wa2, dx1, dy1, dx2, dy2, wa3)
        k3 = (m3 > 0.0) & jnp.logical_not(ov13 | (k2 & ov23))
        ov14 = iou_gt(wx1, wy1, wx2, wy2, wa1, ex1, ey1, ex2, ey2, wa4)
        ov24 = iou_gt(cx1, cy1, cx2, cy2, wa2, ex1, ey1, ex2, ey2, wa4)
        ov34 = iou_gt(dx1, dy1, dx2, dy2, wa3, ex1, ey1, ex2, ey2, wa4)
        k4 = (m4 > 0.0) & jnp.logical_not(ov14 | (k2 & ov24) | (k3 & ov34))
        k2_s = jnp.any(k2)
        k3_s = jnp.any(k3)
        k4_s = jnp.any(k4)

        st = _EMPTY4
        for c in range(_NCH):
            sl = pl.ds(c * _L, _L)
            x1 = x1_v[sl]
            y1 = y1_v[sl]
            x2 = x2_v[sl]
            y2 = y2_v[sl]
            ar = area_v[sl]
            sa = sact_v[sl]
            li = lane + c * _L
            gi = li + _splat_i(base)
            h1 = use1 & (iou_gt(x1, y1, x2, y2, ar, wx1, wy1, wx2, wy2, wa1)
                         | (gi == g1))
            h2 = k2 & (iou_gt(x1, y1, x2, y2, ar, cx1, cy1, cx2, cy2, wa2)
                       | (gi == g2))
            h3 = k3 & (iou_gt(x1, y1, x2, y2, ar, dx1, dy1, dx2, dy2, wa3)
                       | (gi == g3))
            h4 = k4 & (iou_gt(x1, y1, x2, y2, ar, ex1, ey1, ex2, ey2, wa4)
                       | (gi == g4))
            toneg = (use1 & (gi == g1)) | (k2 & (gi == g2)) \
                | (k3 & (gi == g3)) | (k4 & (gi == g4))
            hit = (sa > 0.0) & (h1 | h2 | h3 | h4)
            sa_new = jnp.where(hit, jnp.where(toneg, _NEG, -sa), sa)
            sact_v[sl] = sa_new
            st = acc_top4(sa_new, li, st)

        k2_i = k2_s.astype(jnp.int32)
        k3_i = k3_s.astype(jnp.int32)
        k4_i = k4_s.astype(jnp.int32)
        pos2 = r + 1
        pos3 = r + 1 + k2_i
        pos4 = r + 1 + k2_i + k3_i

        @pl.when(use1_s & (sid == 0))
        def _out1():
            row = build_vreg([(0, wx1), (1, wy1), (2, wx2), (3, wy2), (4, m1)])
            plsc.store_scatter(outbuf_v, [_splat_i(r * _L) + lane], row)

        @pl.when(k2_s & (pos2 < _MAXDET) & (sid == 0))
        def _out2():
            row = build_vreg([(0, cx1), (1, cy1), (2, cx2), (3, cy2), (4, m2)])
            plsc.store_scatter(outbuf_v, [_splat_i(pos2 * _L) + lane], row)

        @pl.when(k3_s & (pos3 < _MAXDET) & (sid == 0))
        def _out3():
            row = build_vreg([(0, dx1), (1, dy1), (2, dx2), (3, dy2), (4, m3)])
            plsc.store_scatter(outbuf_v, [_splat_i(pos3 * _L) + lane], row)

        @pl.when(k4_s & (pos4 < _MAXDET) & (sid == 0))
        def _out4():
            row = build_vreg([(0, ex1), (1, ey1), (2, ex2), (3, ey2), (4, m4)])
            plsc.store_scatter(outbuf_v, [_splat_i(pos4 * _L) + lane], row)

        @pl.when(jnp.logical_not(use1_s))
        def _phase23():
            fmv, fiv = sweep_filler()
            fbx1, fby1, fbx2, fby2, _ = box_at(fiv)
            fgv = (fiv + base).astype(jnp.float32)
            rec = build_vreg([(0, fmv), (1, fgv), (2, fbx1), (3, fby1),
                              (4, fbx2), (5, fby2)])
            stage_v[pl.ds(0, _L)] = rec
            pltpu.sync_copy(
                stage_v.at[pl.ds(0, _L)],
                shared_rec2.at[pl.ds(parity * _FTBL + sid * _L, _L)])
            plsc.subcore_barrier()
            pltpu.sync_copy(shared_rec2.at[pl.ds(parity * _FTBL, _FTBL)],
                            frec_v)
            fcol = plsc.load_gather(frec_v, [lane * _L])
            gfv, widf = lex_reduce(fcol, lane)
            usef = gfv > 0.0

            fo = widf * _L
            fidxv = plsc.load_gather(frec_v, [fo + 1]).astype(jnp.int32)
            ox1 = plsc.load_gather(frec_v, [fo + 2])
            oy1 = plsc.load_gather(frec_v, [fo + 3])
            ox2 = plsc.load_gather(frec_v, [fo + 4])
            oy2 = plsc.load_gather(frec_v, [fo + 5])

            lidxv = fidxv - base
            owner = (lidxv >= 0) & (lidxv < _PER_W)
            lclampv = jnp.clip(lidxv, 0, _PER_W - 1)
            plsc.store_scatter(sact_v, [lclampv], _splat_f(_NEG),
                               mask=(lane == 0) & usef & owner)

            @pl.when(sid == 0)
            def _out23():
                zero = _splat_f(0.0)
                row = build_vreg([
                    (0, jnp.where(usef, ox1, zero)),
                    (1, jnp.where(usef, oy1, zero)),
                    (2, jnp.where(usef, ox2, zero)),
                    (3, jnp.where(usef, oy2, zero)),
                    (4, _splat_f(_NEG))])
                plsc.store_scatter(outbuf_v, [_splat_i(r * _L) + lane], row)

        nst = top4_merge(st)
        dr = jnp.where(use1_s, 1 + k2_i + k3_i + k4_i, 1).astype(jnp.int32)
        return (r + dr, rnd + 1) + nst

    st0 = _EMPTY4
    for c in range(_NCH):
        st0 = acc_top4(sact_v[pl.ds(c * _L, _L)], lane + c * _L, st0)
    init = (jnp.int32(0), jnp.int32(0)) + top4_merge(st0)
    lax.while_loop(lambda c: c[0] < _MAXDET, round_body, init)

    @pl.when((sid == 0) & (cid == 0))
    def _flush():
        pltpu.sync_copy(outbuf_v, out_hbm)


def kernel(boxes, scores):
    n = boxes.shape[0]
    boxes_p = jnp.zeros((_PAD, 4), jnp.float32).at[:n].set(boxes)
    scores_p = jnp.full((_PAD,), -1.0, jnp.float32).at[:n].set(scores)
    cx1, cy1, cx2, cy2 = (boxes_p[:, j] for j in range(4))

    mesh = plsc.VectorSubcoreMesh(core_axis_name="c", subcore_axis_name="s")
    run = functools.partial(
        pl.kernel,
        out_type=jax.ShapeDtypeStruct((_MAXDET * _L,), jnp.float32),
        mesh=mesh,
        compiler_params=pltpu.CompilerParams(needs_layout_passes=False),
        scratch_types=[
            pltpu.VMEM((_PER_W,), jnp.float32),
            pltpu.VMEM((_PER_W,), jnp.float32),
            pltpu.VMEM((_PER_W,), jnp.float32),
            pltpu.VMEM((_PER_W,), jnp.float32),
            pltpu.VMEM((_PER_W,), jnp.float32),
            pltpu.VMEM((_PER_W,), jnp.float32),
            pltpu.VMEM((_RW,), jnp.float32),
            pltpu.VMEM((_TBL,), jnp.float32),
            pltpu.VMEM((_FTBL,), jnp.float32),
            pltpu.VMEM((_MAXDET * _L,), jnp.float32),
            pltpu.VMEM_SHARED((2 * _TBL,), jnp.float32),
            pltpu.VMEM_SHARED((2 * _FTBL,), jnp.float32),
        ],
    )(_nms_body)
    out = run(cx1, cy1, cx2, cy2, scores_p)
    return out.reshape(_MAXDET, _L)[:, :5]
